# baseline (speedup 1.0000x reference)
import functools

import jax
import jax.numpy as jnp
from jax import lax
from jax.experimental import pallas as pl
from jax.experimental.pallas import tpu as pltpu
from jax.experimental.pallas import tpu_sc as plsc

_F_SCORE = 1.2447
_BETA = 0.999
_B = 512
_D = 512
_K = 100
_NW = 32
_SPT = _B // _NW
_L = 16


def _sc_z_kernel(labels_hbm, cmean_hbm, features_hbm, z_hbm,
                 lab_v, rows_v, m_v, t_v, z_v, sem_f, sem_m):
    wid = lax.axis_index("s") * 2 + lax.axis_index("c")
    base = wid * _SPT

    pltpu.sync_copy(labels_hbm.at[pl.ds(base, _SPT)], lab_v)
    fcp = pltpu.async_copy(features_hbm.at[pl.ds(base, _SPT)], rows_v, sem_f)
    mcp = pltpu.async_copy(cmean_hbm.at[lab_v], m_v, sem_m)
    fcp.wait()
    mcp.wait()

    lane = jnp.arange(_L, dtype=jnp.int32)

    def body(j, carry):
        jv = jnp.full((_L,), j, dtype=jnp.int32)
        acc = jnp.zeros((_L,), jnp.float32)
        for q in range(_D // _L):
            fv = rows_v[j, pl.ds(q * _L, _L)]
            mv = m_v[j, pl.ds(q * _L, _L)]
            t = jnp.abs(fv - mv)
            acc = acc + jnp.where(fv != 0.0, t, 0.0)
        plsc.store_scatter(t_v, [lane, jv], acc)
        return carry

    lax.fori_loop(0, _SPT, body, 0, unroll=_SPT)

    zg = t_v[0, pl.ds(0, _L)]
    for r in range(1, _L):
        zg = zg + t_v[r, pl.ds(0, _L)]
    z_v[pl.ds(0, _L)] = zg
    pltpu.sync_copy(z_v, z_hbm.at[pl.ds(base, _SPT)])


_sc_z = functools.partial(
    pl.kernel,
    out_type=jax.ShapeDtypeStruct((_B,), jnp.float32),
    mesh=plsc.VectorSubcoreMesh(core_axis_name="c", subcore_axis_name="s",
                                num_cores=2, num_subcores=_NW // 2),
    compiler_params=pltpu.CompilerParams(needs_layout_passes=False),
    scratch_types=[
        pltpu.VMEM((_SPT,), jnp.int32),
        pltpu.VMEM((_SPT, _D), jnp.float32),
        pltpu.VMEM((_SPT, _D), jnp.float32),
        pltpu.VMEM((_L, _SPT), jnp.float32),
        pltpu.VMEM((_SPT,), jnp.float32),
        pltpu.SemaphoreType.DMA,
        pltpu.SemaphoreType.DMA,
    ],
)(_sc_z_kernel)


def _tc_pre_kernel(labels_ref, counts_ref, features_ref, cmean_ref):
    lab_row = labels_ref[:].reshape(1, -1)
    counts_col = counts_ref[:].reshape(-1, 1)
    b_sz = lab_row.shape[1]
    k = counts_col.shape[0]

    oht = (lab_row == jax.lax.broadcasted_iota(jnp.int32, (k, b_sz), 0)
           ).astype(jnp.float32)
    f = features_ref[:]
    segsum = jnp.dot(oht, f, preferred_element_type=jnp.float32)
    cmean_ref[:, :] = segsum / counts_col


def _tc_post_kernel(logits_ref, labels_ref, counts_ref, z_ref, out_ref):
    lab_row = labels_ref[:].reshape(1, -1)
    counts_col = counts_ref[:].reshape(-1, 1)
    b_sz = lab_row.shape[1]
    k = counts_col.shape[0]

    lab_col = lab_row.T
    oh = (lab_col == jax.lax.broadcasted_iota(jnp.int32, (b_sz, k), 1)
          ).astype(jnp.float32)
    oht = (lab_row == jax.lax.broadcasted_iota(jnp.int32, (k, b_sz), 0)
           ).astype(jnp.float32)

    inv_counts = 1.0 / counts_col
    z = z_ref[:].reshape(1, -1).T

    s = jnp.dot(oht, z, preferred_element_type=jnp.float32)
    zi_mean = s * inv_counts
    z_mean = jnp.sum(zi_mean) / k
    n_total = jnp.sum(counts_col)

    zi_g = jnp.dot(oh, zi_mean, preferred_element_type=jnp.float32)
    ssw = jnp.sum((z - zi_g) ** 2 *
                  (z != 0.0).astype(jnp.float32)) / (n_total - k)
    sb = (zi_mean - z_mean) ** 2 * counts_col
    ssb = jnp.sum(sb) / (k - 1)

    cq = _F_SCORE * ssw * (k - 1) - (ssb * (k - 1) - sb)
    a = z_mean ** 2
    b = -(2.0 * z_mean * s + cq)
    cc = s ** 2
    disc = jnp.sqrt(b * b - 4.0 * a * cc)
    n_lb = jnp.abs((-b - disc) / (2.0 * a))
    n_ub = jnp.abs((-b + disc) / (2.0 * a))

    beta = jnp.where(
        counts_col < n_lb,
        jnp.power(_BETA, 1.0 / (n_lb - counts_col)),
        jnp.where(counts_col > n_ub,
                  jnp.power(_BETA, 1.0 / (counts_col - n_ub)),
                  _BETA))
    eff = 1.0 - jnp.power(beta, counts_col)
    w_cls = (1.0 - beta) / eff
    w_cls = w_cls / jnp.sum(w_cls) * k
    w_n = jnp.dot(oh, w_cls, preferred_element_type=jnp.float32)

    lg = logits_ref[:]
    mx = jnp.max(lg, axis=1, keepdims=True)
    e = jnp.exp(lg - mx)
    pred = e / jnp.sum(e, axis=1, keepdims=True)
    log_p = jnp.maximum(jnp.log(pred), -100.0)
    log_1mp = jnp.maximum(jnp.log(1.0 - pred), -100.0)
    bce = -(oh * log_p + (1.0 - oh) * log_1mp)
    total = jnp.sum(w_n * bce, axis=None, keepdims=True)
    out_ref[:, :] = total / (b_sz * k)


def kernel(logits, labels, features, sample_num_per_cls):
    lab = labels.astype(jnp.int32)
    cmean = pl.pallas_call(
        _tc_pre_kernel,
        out_shape=jax.ShapeDtypeStruct((_K, _D), jnp.float32),
    )(lab, sample_num_per_cls, features)
    z = _sc_z(lab, cmean, features)
    out = pl.pallas_call(
        _tc_post_kernel,
        out_shape=jax.ShapeDtypeStruct((1, 1), jnp.float32),
    )(logits, lab, sample_num_per_cls, z)
    return out[0, 0]


# speedup vs baseline: 4.4003x; 4.4003x over previous
import jax
import jax.numpy as jnp
from jax.experimental import pallas as pl
from jax.experimental.pallas import tpu as pltpu

_F_SCORE = 1.2447
_BETA = 0.999


def _homovar_kernel(logits_ref, labels_ref, counts_ref, features_hbm,
                    out_ref, f_vmem, sem_a, sem_b):
    d = f_vmem.shape[1]
    dh = d // 2
    cp_a = pltpu.make_async_copy(features_hbm.at[:, pl.ds(0, dh)],
                                 f_vmem.at[:, pl.ds(0, dh)], sem_a)
    cp_b = pltpu.make_async_copy(features_hbm.at[:, pl.ds(dh, dh)],
                                 f_vmem.at[:, pl.ds(dh, dh)], sem_b)
    cp_a.start()
    cp_b.start()

    lab_row = labels_ref[:].reshape(1, -1)
    counts_row = counts_ref[:].reshape(1, -1)
    b_sz = lab_row.shape[1]
    k = counts_row.shape[1]

    lab_col = lab_row.T
    oh = (lab_col == jax.lax.broadcasted_iota(jnp.int32, (b_sz, k), 1)
          ).astype(jnp.float32)
    oht = (lab_row == jax.lax.broadcasted_iota(jnp.int32, (k, b_sz), 0)
           ).astype(jnp.float32)
    inv_counts = 1.0 / counts_row
    ones_k = jnp.ones((k, 1), jnp.float32)

    lg = logits_ref[:]
    mx = jnp.max(lg, axis=1, keepdims=True)
    e = jnp.exp(lg - mx)
    se = jnp.sum(e, axis=1, keepdims=True)
    pred = e / se
    log_p = jnp.maximum(lg - mx - jnp.log(se), -100.0)
    log_1mp = jnp.maximum(jnp.log(1.0 - pred), -100.0)
    bce = oh * (log_1mp - log_p) - log_1mp
    bcesum = jnp.dot(bce, ones_k,
                     preferred_element_type=jnp.float32)

    ohm = oht * inv_counts.T
    ones_h = jnp.ones((dh, 1), jnp.float32)
    z = None
    for half, cp in enumerate((cp_a, cp_b)):
        cp.wait()
        fh = f_vmem[:, pl.ds(half * dh, dh)]
        cmean = jnp.dot(ohm, fh, preferred_element_type=jnp.float32)
        mh = jnp.dot(oh, cmean, preferred_element_type=jnp.float32)
        th = jnp.abs(fh - mh) * (fh != 0.0).astype(jnp.float32)
        zh = jnp.dot(th, ones_h, preferred_element_type=jnp.float32)
        z = zh if z is None else z + zh

    zmask = (z != 0.0).astype(jnp.float32)
    zz = jnp.concatenate([z, zmask], axis=1)
    y = jax.lax.dot_general(zz, oht, (((0,), (1,)), ((), ())),
                            preferred_element_type=jnp.float32)
    s = y[0:1, :]
    nz = y[1:2, :]

    zi_mean = s * inv_counts
    z_mean = jnp.sum(zi_mean) / k
    n_total = jnp.sum(counts_row)

    ssw = (jnp.sum(z * z) - 2.0 * jnp.sum(zi_mean * s)
           + jnp.sum(zi_mean * zi_mean * nz)) / (n_total - k)
    sb = (zi_mean - z_mean) ** 2 * counts_row
    ssb = jnp.sum(sb) / (k - 1)

    cq = _F_SCORE * ssw * (k - 1) - (ssb * (k - 1) - sb)
    a = z_mean ** 2
    b = -(2.0 * z_mean * s + cq)
    cc = s ** 2
    disc = jnp.sqrt(b * b - 4.0 * a * cc)
    n_lb = jnp.abs((-b - disc) / (2.0 * a))
    n_ub = jnp.abs((-b + disc) / (2.0 * a))

    beta = jnp.where(
        counts_row < n_lb,
        jnp.power(_BETA, 1.0 / (n_lb - counts_row)),
        jnp.where(counts_row > n_ub,
                  jnp.power(_BETA, 1.0 / (counts_row - n_ub)),
                  _BETA))
    eff = 1.0 - jnp.power(beta, counts_row)
    w_cls = (1.0 - beta) / eff
    w_cls = w_cls / jnp.sum(w_cls) * k
    w_n = jnp.dot(w_cls, oht,
                  preferred_element_type=jnp.float32)

    total = jax.lax.dot_general(w_n, bcesum, (((1,), (0,)), ((), ())),
                                preferred_element_type=jnp.float32)
    out_ref[:, :] = total / (b_sz * k)


def kernel(logits, labels, features, sample_num_per_cls):
    bsz, d = features.shape
    out = pl.pallas_call(
        _homovar_kernel,
        in_specs=[
            pl.BlockSpec(memory_space=pltpu.VMEM),
            pl.BlockSpec(memory_space=pltpu.VMEM),
            pl.BlockSpec(memory_space=pltpu.VMEM),
            pl.BlockSpec(memory_space=pltpu.HBM),
        ],
        scratch_shapes=[
            pltpu.VMEM((bsz, d), jnp.float32),
            pltpu.SemaphoreType.DMA,
            pltpu.SemaphoreType.DMA,
        ],
        out_shape=jax.ShapeDtypeStruct((1, 1), jnp.float32),
    )(logits, labels.astype(jnp.int32), sample_num_per_cls, features)
    return out[0, 0]


# speedup vs baseline: 4.9174x; 1.1175x over previous
import jax
import jax.numpy as jnp
from jax.experimental import pallas as pl
from jax.experimental.pallas import tpu as pltpu

_F_SCORE = 1.2447
_BETA = 0.999


def _homovar_kernel(logits_ref, labels_ref, counts_ref, features_ref,
                    out_ref):

    lab_row = labels_ref[:].reshape(1, -1)
    counts_row = counts_ref[:].reshape(1, -1)
    b_sz = lab_row.shape[1]
    k = counts_row.shape[1]

    lab_col = lab_row.T
    oh = (lab_col == jax.lax.broadcasted_iota(jnp.int32, (b_sz, k), 1)
          ).astype(jnp.float32)
    oht = (lab_row == jax.lax.broadcasted_iota(jnp.int32, (k, b_sz), 0)
           ).astype(jnp.float32)
    inv_counts = 1.0 / counts_row
    ones_k = jnp.ones((k, 1), jnp.float32)

    lg = logits_ref[:]
    mx = jnp.max(lg, axis=1, keepdims=True)
    e = jnp.exp(lg - mx)
    se = jnp.sum(e, axis=1, keepdims=True)
    pred = e / se
    log_p = jnp.maximum(lg - mx - jnp.log(se), -100.0)
    log_1mp = jnp.maximum(jnp.log(1.0 - pred), -100.0)
    bce = oh * (log_1mp - log_p) - log_1mp
    bcesum = jnp.dot(bce, ones_k,
                     preferred_element_type=jnp.float32)

    f = features_ref[:]
    ohm = oht * inv_counts.T
    cmean = jnp.dot(ohm, f, preferred_element_type=jnp.float32)
    m = jnp.dot(oh, cmean, preferred_element_type=jnp.float32)
    t = jnp.abs(f - m) * (f != 0.0).astype(jnp.float32)
    ones_d = jnp.ones((f.shape[1], 1), jnp.float32)
    z = jnp.dot(t, ones_d, preferred_element_type=jnp.float32)

    zmask = (z != 0.0).astype(jnp.float32)
    zz = jnp.concatenate([z, zmask], axis=1)
    y = jax.lax.dot_general(zz, oht, (((0,), (1,)), ((), ())),
                            preferred_element_type=jnp.float32)
    s = y[0:1, :]
    nz = y[1:2, :]

    zi_mean = s * inv_counts
    z_mean = jnp.sum(zi_mean) / k
    n_total = jnp.sum(counts_row)

    ssw = (jnp.sum(z * z) - 2.0 * jnp.sum(zi_mean * s)
           + jnp.sum(zi_mean * zi_mean * nz)) / (n_total - k)
    sb = (zi_mean - z_mean) ** 2 * counts_row
    ssb = jnp.sum(sb) / (k - 1)

    cq = _F_SCORE * ssw * (k - 1) - (ssb * (k - 1) - sb)
    a = z_mean ** 2
    b = -(2.0 * z_mean * s + cq)
    cc = s ** 2
    disc = jnp.sqrt(b * b - 4.0 * a * cc)
    n_lb = jnp.abs((-b - disc) / (2.0 * a))
    n_ub = jnp.abs((-b + disc) / (2.0 * a))

    beta = jnp.where(
        counts_row < n_lb,
        jnp.power(_BETA, 1.0 / (n_lb - counts_row)),
        jnp.where(counts_row > n_ub,
                  jnp.power(_BETA, 1.0 / (counts_row - n_ub)),
                  _BETA))
    eff = 1.0 - jnp.power(beta, counts_row)
    w_cls = (1.0 - beta) / eff
    w_cls = w_cls / jnp.sum(w_cls) * k
    w_n = jnp.dot(w_cls, oht,
                  preferred_element_type=jnp.float32)

    total = jax.lax.dot_general(w_n, bcesum, (((1,), (0,)), ((), ())),
                                preferred_element_type=jnp.float32)
    out_ref[:, :] = total / (b_sz * k)


def kernel(logits, labels, features, sample_num_per_cls):
    out = pl.pallas_call(
        _homovar_kernel,
        out_shape=jax.ShapeDtypeStruct((1, 1), jnp.float32),
    )(logits, labels.astype(jnp.int32), sample_num_per_cls, features)
    return out[0, 0]
